# baseline (speedup 1.0000x reference)
import functools

import jax
import jax.numpy as jnp
from jax import lax
from jax.experimental import pallas as pl
from jax.experimental.pallas import tpu as pltpu
from jax.experimental.pallas import tpu_sc as plsc

D_IN = 128
D_AUG = 144
CHUNK = 128
NC = 2
NS = 16
NW = NC * NS


def _sc_aggregate(chunks_per_worker, n_pad):
    mesh = plsc.VectorSubcoreMesh(core_axis_name="c", subcore_axis_name="s")
    rows_per_tile = n_pad // NS
    cw = chunks_per_worker
    assert cw % 2 == 0
    slice_chunks = [CHUNK] * (rows_per_tile // CHUNK)
    if rows_per_tile % CHUNK:
        slice_chunks.append(rows_per_tile % CHUNK)

    @functools.partial(
        pl.kernel,
        mesh=mesh,
        compiler_params=pltpu.CompilerParams(use_tc_tiling_on_sc=False),
        out_type=jax.ShapeDtypeStruct((NC, n_pad, D_AUG), jnp.float32),
        scratch_types=[
            pltpu.VMEM((CHUNK,), jnp.int32),
            pltpu.VMEM((CHUNK,), jnp.int32),
            pltpu.VMEM((CHUNK,), jnp.int32),
            pltpu.VMEM((CHUNK,), jnp.int32),
            pltpu.VMEM((CHUNK, D_AUG), jnp.float32),
            pltpu.VMEM((CHUNK, D_AUG), jnp.float32),
            pltpu.VMEM_SHARED((n_pad, D_AUG), jnp.float32),
            pltpu.SemaphoreType.DMA,
            pltpu.SemaphoreType.DMA,
            pltpu.SemaphoreType.DMA,
            pltpu.SemaphoreType.DMA,
        ],
    )
    def sc_agg(xa_hbm, src_hbm, dst_hbm, out_hbm, is_a, id_a, is_b, id_b,
               buf_a, buf_b, acc_sh, sem_ia, sem_ib, sem_a, sem_b):
        c = lax.axis_index("c")
        s = lax.axis_index("s")
        wid = s * NC + c
        row0 = wid * cw

        def zero_row(i, carry):
            for j in range(D_AUG // 16):
                buf_a[i, pl.ds(j * 16, 16)] = jnp.zeros((16,), jnp.float32)
            return carry

        lax.fori_loop(0, CHUNK, zero_row, 0)
        r0 = s * rows_per_tile
        for sz in slice_chunks:
            pltpu.sync_copy(buf_a.at[pl.ds(0, sz)], acc_sh.at[pl.ds(r0, sz)])
            r0 += sz
        plsc.subcore_barrier()

        slot = ((is_a, id_a, sem_ia, buf_a, sem_a),
                (is_b, id_b, sem_ib, buf_b, sem_b))

        def idx_start(j, k):
            is_x, id_x, sem_ix, _, _ = slot[k]
            pltpu.make_async_copy(src_hbm.at[row0 + j], is_x, sem_ix).start()
            pltpu.make_async_copy(dst_hbm.at[row0 + j], id_x, sem_ix).start()

        def idx_wait(j, k):
            is_x, id_x, sem_ix, _, _ = slot[k]
            pltpu.make_async_copy(src_hbm.at[row0 + j], is_x, sem_ix).wait()
            pltpu.make_async_copy(dst_hbm.at[row0 + j], id_x, sem_ix).wait()

        def gather_start(k):
            is_x, _, _, buf, sem = slot[k]
            pltpu.make_async_copy(xa_hbm.at[is_x], buf, sem).start()

        def gather_wait_scatter(k):
            is_x, id_x, _, buf, sem = slot[k]
            pltpu.make_async_copy(xa_hbm.at[is_x], buf, sem).wait()
            pltpu.sync_copy(buf, acc_sh.at[id_x], add=True)

        def step(j, cur, prefetch):
            nxt = 1 - cur
            idx_wait(j + 1, nxt)
            gather_start(nxt)
            gather_wait_scatter(cur)
            if prefetch:
                idx_start(j + 2, cur)

        idx_start(0, 0)
        idx_start(1, 1)
        idx_wait(0, 0)
        gather_start(0)

        def body(i, carry):
            step(2 * i, 0, True)
            step(2 * i + 1, 1, True)
            return carry

        lax.fori_loop(0, cw // 2 - 1, body, 0)
        step(cw - 2, 0, False)
        gather_wait_scatter(1)
        plsc.subcore_barrier()

        r0 = s * rows_per_tile
        for sz in slice_chunks:
            pltpu.sync_copy(acc_sh.at[pl.ds(r0, sz)], buf_a.at[pl.ds(0, sz)])
            pltpu.sync_copy(buf_a.at[pl.ds(0, sz)],
                            out_hbm.at[c, pl.ds(r0, sz)])
            r0 += sz

    return sc_agg


def _tc_combine(x, agg2, W, b, n_nodes):
    blk = 1000

    def body(x_ref, a_ref, w_ref, b_ref, o_ref):
        ssum = a_ref[0] + a_ref[1]
        cnt = ssum[:, D_IN:D_IN + 1]
        agg = ssum[:, :D_IN] / (cnt + 1e-8)
        h = x_ref[...] + agg
        o_ref[...] = lax.dot_general(
            h, w_ref[...], (((1,), (1,)), ((), ())),
            preferred_element_type=jnp.float32) + b_ref[...]

    n_pad = agg2.shape[1]
    return pl.pallas_call(
        body,
        grid=(n_nodes // blk,),
        in_specs=[
            pl.BlockSpec((blk, D_IN), lambda i: (i, 0)),
            pl.BlockSpec((NC, blk, D_AUG), lambda i: (0, i, 0)),
            pl.BlockSpec((D_IN, D_IN), lambda i: (0, 0)),
            pl.BlockSpec((1, D_IN), lambda i: (0, 0)),
        ],
        out_specs=pl.BlockSpec((blk, D_IN), lambda i: (i, 0)),
        out_shape=jax.ShapeDtypeStruct((n_nodes, D_IN), jnp.float32),
    )(x, agg2, W, b.reshape(1, D_IN))


def kernel(x, edge_index, W, b):
    n = x.shape[0]
    e = edge_index.shape[1]
    grain = 2 * NW * CHUNK
    e_pad = ((e + grain - 1) // grain) * grain
    chunks_per_worker = e_pad // (NW * CHUNK)
    n_pad = ((n + 1 + NS - 1) // NS) * NS

    src = edge_index[0].astype(jnp.int32)
    dst = edge_index[1].astype(jnp.int32)
    pad = e_pad - e
    if pad:
        src = jnp.concatenate([src, jnp.zeros((pad,), jnp.int32)])
        dst = jnp.concatenate([dst, jnp.full((pad,), n, jnp.int32)])
    src = src.reshape(e_pad // CHUNK, CHUNK)
    dst = dst.reshape(e_pad // CHUNK, CHUNK)
    xa = jnp.concatenate(
        [x, jnp.ones((n, D_AUG - D_IN), x.dtype)], axis=1)

    agg2 = _sc_aggregate(chunks_per_worker, n_pad)(xa, src, dst)
    return _tc_combine(x, agg2, W, b, n)


# speedup vs baseline: 1.0400x; 1.0400x over previous
import functools

import jax
import jax.numpy as jnp
from jax import lax
from jax.experimental import pallas as pl
from jax.experimental.pallas import tpu as pltpu
from jax.experimental.pallas import tpu_sc as plsc

D_IN = 128
D_AUG = 144
CHUNK = 64
NSLOT = 4
NC = 2
NS = 16
NW = NC * NS


def _sc_aggregate(cw0, cw1, n_pad):
    mesh = plsc.VectorSubcoreMesh(core_axis_name="c", subcore_axis_name="s")
    rows_per_tile = n_pad // NS
    assert cw0 % NSLOT == 0 and cw1 % NSLOT == 0
    slice_chunks = [2 * CHUNK] * (rows_per_tile // (2 * CHUNK))
    if rows_per_tile % (2 * CHUNK):
        slice_chunks.append(rows_per_tile % (2 * CHUNK))

    scratch = ([pltpu.VMEM((CHUNK,), jnp.int32)] * (2 * NSLOT)
               + [pltpu.VMEM((CHUNK, D_AUG), jnp.float32)] * NSLOT
               + [pltpu.VMEM_SHARED((n_pad, D_AUG), jnp.float32)]
               + [pltpu.SemaphoreType.DMA] * (2 * NSLOT))

    @functools.partial(
        pl.kernel,
        mesh=mesh,
        compiler_params=pltpu.CompilerParams(use_tc_tiling_on_sc=False),
        out_type=jax.ShapeDtypeStruct((NC, n_pad, D_AUG), jnp.float32),
        scratch_types=scratch,
    )
    def sc_agg(xa_hbm, src_hbm, dst_hbm, out_hbm, *refs):
        idx_refs = refs[:2 * NSLOT]
        bufs = refs[2 * NSLOT:3 * NSLOT]
        acc_sh = refs[3 * NSLOT]
        sems = refs[3 * NSLOT + 1:]
        c = lax.axis_index("c")
        s = lax.axis_index("s")
        ncw = jnp.where(c == 0, cw0, cw1)
        row0 = jnp.where(c == 0, s * cw0, NS * cw0 + s * cw1)

        def zero_row(i, carry):
            for j in range(D_AUG // 16):
                bufs[0][i, pl.ds(j * 16, 16)] = jnp.zeros((16,), jnp.float32)
                bufs[1][i, pl.ds(j * 16, 16)] = jnp.zeros((16,), jnp.float32)
            return carry

        lax.fori_loop(0, CHUNK, zero_row, 0)
        r0 = s * rows_per_tile
        for sz in slice_chunks:
            if sz > CHUNK:
                pltpu.sync_copy(bufs[0], acc_sh.at[pl.ds(r0, CHUNK)])
                pltpu.sync_copy(bufs[1].at[pl.ds(0, sz - CHUNK)],
                                acc_sh.at[pl.ds(r0 + CHUNK, sz - CHUNK)])
            else:
                pltpu.sync_copy(bufs[0].at[pl.ds(0, sz)],
                                acc_sh.at[pl.ds(r0, sz)])
            r0 += sz
        plsc.subcore_barrier()

        def idx_start(j, k):
            pltpu.make_async_copy(src_hbm.at[row0 + j], idx_refs[2 * k],
                                  sems[2 * k]).start()
            pltpu.make_async_copy(dst_hbm.at[row0 + j], idx_refs[2 * k + 1],
                                  sems[2 * k]).start()

        def idx_wait(j, k):
            pltpu.make_async_copy(src_hbm.at[row0 + j], idx_refs[2 * k],
                                  sems[2 * k]).wait()
            pltpu.make_async_copy(dst_hbm.at[row0 + j], idx_refs[2 * k + 1],
                                  sems[2 * k]).wait()

        def gather_start(k):
            pltpu.make_async_copy(xa_hbm.at[idx_refs[2 * k]], bufs[k],
                                  sems[2 * k + 1]).start()

        def gather_wait_scatter(k):
            pltpu.make_async_copy(xa_hbm.at[idx_refs[2 * k]], bufs[k],
                                  sems[2 * k + 1]).wait()
            pltpu.sync_copy(bufs[k], acc_sh.at[idx_refs[2 * k + 1]],
                            add=True)

        def step(j, k, prefetch):
            km1 = (k - 1) % NSLOT
            idx_wait(j + NSLOT - 1, km1)
            gather_start(km1)
            gather_wait_scatter(k)
            if prefetch:
                idx_start(j + NSLOT, k)

        for k in range(NSLOT):
            idx_start(k, k)
        for k in range(NSLOT - 1):
            idx_wait(k, k)
            gather_start(k)

        def body(i, carry):
            for k in range(NSLOT):
                step(NSLOT * i + k, k, True)
            return carry

        lax.fori_loop(0, ncw // NSLOT - 1, body, 0)
        for k in range(NSLOT):
            j = ncw - NSLOT + k
            if k + NSLOT - 1 < NSLOT:
                km1 = (k - 1) % NSLOT
                idx_wait(j + NSLOT - 1, km1)
                gather_start(km1)
            gather_wait_scatter(k)
        plsc.subcore_barrier()

        r0 = s * rows_per_tile
        for sz in slice_chunks:
            csz = min(sz, 2 * CHUNK)
            pltpu.sync_copy(acc_sh.at[pl.ds(r0, CHUNK)], bufs[0])
            pltpu.sync_copy(acc_sh.at[pl.ds(r0 + CHUNK, csz - CHUNK)],
                            bufs[1].at[pl.ds(0, csz - CHUNK)])
            pltpu.sync_copy(bufs[0], out_hbm.at[c, pl.ds(r0, CHUNK)])
            pltpu.sync_copy(bufs[1].at[pl.ds(0, csz - CHUNK)],
                            out_hbm.at[c, pl.ds(r0 + CHUNK, csz - CHUNK)])
            r0 += sz

    return sc_agg


def _tc_combine(x, agg2, W, b, n_nodes):
    blk = 1000

    def body(x_ref, a_ref, w_ref, b_ref, o_ref):
        ssum = a_ref[0] + a_ref[1]
        cnt = ssum[:, D_IN:D_IN + 1]
        agg = ssum[:, :D_IN] / (cnt + 1e-8)
        h = x_ref[...] + agg
        o_ref[...] = lax.dot_general(
            h, w_ref[...], (((1,), (1,)), ((), ())),
            preferred_element_type=jnp.float32) + b_ref[...]

    return pl.pallas_call(
        body,
        grid=(n_nodes // blk,),
        in_specs=[
            pl.BlockSpec((blk, D_IN), lambda i: (i, 0)),
            pl.BlockSpec((NC, blk, D_AUG), lambda i: (0, i, 0)),
            pl.BlockSpec((D_IN, D_IN), lambda i: (0, 0)),
            pl.BlockSpec((1, D_IN), lambda i: (0, 0)),
        ],
        out_specs=pl.BlockSpec((blk, D_IN), lambda i: (i, 0)),
        out_shape=jax.ShapeDtypeStruct((n_nodes, D_IN), jnp.float32),
    )(x, agg2, W, b.reshape(1, D_IN))


def kernel(x, edge_index, W, b):
    n = x.shape[0]
    e = edge_index.shape[1]
    t = ((e + NS * CHUNK - 1) // (NS * CHUNK) + 2 * NSLOT - 1) \
        // (2 * NSLOT) * (2 * NSLOT)
    cw0 = max(NSLOT, int(round(t * 0.7375 / NSLOT)) * NSLOT)
    cw1 = t - cw0
    e_pad = NS * (cw0 + cw1) * CHUNK
    n_pad = ((n + 1 + NS - 1) // NS) * NS

    src = edge_index[0].astype(jnp.int32)
    dst = edge_index[1].astype(jnp.int32)
    pad = e_pad - e
    if pad:
        src = jnp.concatenate([src, jnp.zeros((pad,), jnp.int32)])
        dst = jnp.concatenate([dst, jnp.full((pad,), n, jnp.int32)])
    src = src.reshape(e_pad // CHUNK, CHUNK)
    dst = dst.reshape(e_pad // CHUNK, CHUNK)
    xa = jnp.concatenate(
        [x, jnp.ones((n, D_AUG - D_IN), x.dtype)], axis=1)

    agg2 = _sc_aggregate(cw0, cw1, n_pad)(xa, src, dst)
    return _tc_combine(x, agg2, W, b, n)


# speedup vs baseline: 1.3746x; 1.3217x over previous
import functools

import jax
import jax.numpy as jnp
from jax import lax
from jax.experimental import pallas as pl
from jax.experimental.pallas import tpu as pltpu
from jax.experimental.pallas import tpu_sc as plsc

D_IN = 128
D_HALF = 64
D_SC = 80
CHUNK = 64
NSLOT = 4
NC = 2
NS = 16
NW = NC * NS


def _chunk_sizes(total, step):
    out = [step] * (total // step)
    if total % step:
        out.append(total % step)
    return out


def _sc_aggregate(chunks_per_tile, n_tab, n_pad):
    mesh = plsc.VectorSubcoreMesh(core_axis_name="c", subcore_axis_name="s")
    cw = chunks_per_tile
    assert cw % NSLOT == 0
    tab_rows = _chunk_sizes(n_tab // NS, CHUNK)
    acc_rows = _chunk_sizes(n_pad // NS, CHUNK)

    scratch = ([pltpu.VMEM((CHUNK,), jnp.int32)] * (2 * NSLOT)
               + [pltpu.VMEM((CHUNK, D_SC), jnp.float32)] * NSLOT
               + [pltpu.VMEM_SHARED((n_tab, D_SC), jnp.float32)]
               + [pltpu.VMEM_SHARED((n_pad, D_SC), jnp.float32)]
               + [pltpu.SemaphoreType.DMA] * (2 * NSLOT))

    @functools.partial(
        pl.kernel,
        mesh=mesh,
        compiler_params=pltpu.CompilerParams(use_tc_tiling_on_sc=False),
        out_type=jax.ShapeDtypeStruct((NC, n_pad, D_SC), jnp.float32),
        scratch_types=scratch,
    )
    def sc_agg(xh_hbm, src_hbm, dst_hbm, out_hbm, *refs):
        idx_refs = refs[:2 * NSLOT]
        bufs = refs[2 * NSLOT:3 * NSLOT]
        tab_sh = refs[3 * NSLOT]
        acc_sh = refs[3 * NSLOT + 1]
        sems = refs[3 * NSLOT + 2:]
        c = lax.axis_index("c")
        s = lax.axis_index("s")
        row0 = s * cw

        def zero_row(i, carry):
            for j in range(D_SC // 16):
                bufs[0][i, pl.ds(j * 16, 16)] = jnp.zeros((16,), jnp.float32)
            return carry

        lax.fori_loop(0, CHUNK, zero_row, 0)
        r0 = s * (n_pad // NS)
        for sz in acc_rows:
            pltpu.sync_copy(bufs[0].at[pl.ds(0, sz)],
                            acc_sh.at[pl.ds(r0, sz)])
            r0 += sz
        r0 = s * (n_tab // NS)
        for sz in tab_rows:
            pltpu.sync_copy(xh_hbm.at[c, pl.ds(r0, sz)],
                            bufs[1].at[pl.ds(0, sz)])
            pltpu.sync_copy(bufs[1].at[pl.ds(0, sz)],
                            tab_sh.at[pl.ds(r0, sz)])
            r0 += sz
        plsc.subcore_barrier()

        def idx_start(j, k):
            pltpu.make_async_copy(src_hbm.at[row0 + j], idx_refs[2 * k],
                                  sems[2 * k]).start()
            pltpu.make_async_copy(dst_hbm.at[row0 + j], idx_refs[2 * k + 1],
                                  sems[2 * k]).start()

        def idx_wait(j, k):
            pltpu.make_async_copy(src_hbm.at[row0 + j], idx_refs[2 * k],
                                  sems[2 * k]).wait()
            pltpu.make_async_copy(dst_hbm.at[row0 + j], idx_refs[2 * k + 1],
                                  sems[2 * k]).wait()

        def gather_start(k):
            pltpu.make_async_copy(tab_sh.at[idx_refs[2 * k]], bufs[k],
                                  sems[2 * k + 1]).start()

        def gather_wait_scatter(k):
            pltpu.make_async_copy(tab_sh.at[idx_refs[2 * k]], bufs[k],
                                  sems[2 * k + 1]).wait()
            pltpu.sync_copy(bufs[k], acc_sh.at[idx_refs[2 * k + 1]],
                            add=True)

        def step(j, k, prefetch):
            km1 = (k - 1) % NSLOT
            idx_wait(j + NSLOT - 1, km1)
            gather_start(km1)
            gather_wait_scatter(k)
            if prefetch:
                idx_start(j + NSLOT, k)

        for k in range(NSLOT):
            idx_start(k, k)
        for k in range(NSLOT - 1):
            idx_wait(k, k)
            gather_start(k)

        def body(i, carry):
            for k in range(NSLOT):
                step(NSLOT * i + k, k, True)
            return carry

        lax.fori_loop(0, cw // NSLOT - 1, body, 0)
        for k in range(NSLOT):
            j = cw - NSLOT + k
            if k == 0:
                km1 = (k - 1) % NSLOT
                idx_wait(j + NSLOT - 1, km1)
                gather_start(km1)
            gather_wait_scatter(k)
        plsc.subcore_barrier()

        r0 = s * (n_pad // NS)
        for sz in acc_rows:
            pltpu.sync_copy(acc_sh.at[pl.ds(r0, sz)],
                            bufs[0].at[pl.ds(0, sz)])
            pltpu.sync_copy(bufs[0].at[pl.ds(0, sz)],
                            out_hbm.at[c, pl.ds(r0, sz)])
            r0 += sz

    return sc_agg


def _tc_combine(x, agg2, W, b, n_nodes):
    blk = 1000

    def body(x_ref, a_ref, w_ref, b_ref, o_ref):
        s0 = a_ref[0]
        s1 = a_ref[1]
        cnt = s0[:, D_HALF:D_HALF + 1]
        ssum = jnp.concatenate([s0[:, :D_HALF], s1[:, :D_HALF]], axis=1)
        agg = ssum / (cnt + 1e-8)
        h = x_ref[...] + agg
        o_ref[...] = lax.dot_general(
            h, w_ref[...], (((1,), (1,)), ((), ())),
            preferred_element_type=jnp.float32) + b_ref[...]

    return pl.pallas_call(
        body,
        grid=(n_nodes // blk,),
        in_specs=[
            pl.BlockSpec((blk, D_IN), lambda i: (i, 0)),
            pl.BlockSpec((NC, blk, D_SC), lambda i: (0, i, 0)),
            pl.BlockSpec((D_IN, D_IN), lambda i: (0, 0)),
            pl.BlockSpec((1, D_IN), lambda i: (0, 0)),
        ],
        out_specs=pl.BlockSpec((blk, D_IN), lambda i: (i, 0)),
        out_shape=jax.ShapeDtypeStruct((n_nodes, D_IN), jnp.float32),
    )(x, agg2, W, b.reshape(1, D_IN))


def kernel(x, edge_index, W, b):
    n = x.shape[0]
    e = edge_index.shape[1]
    grain = NS * CHUNK * NSLOT
    e_pad = ((e + grain - 1) // grain) * grain
    chunks_per_tile = e_pad // (NS * CHUNK)
    n_pad = ((n + NS - 1) // NS + 1) * NS

    src = edge_index[0].astype(jnp.int32)
    dst = edge_index[1].astype(jnp.int32)
    pad = e_pad - e
    if pad:
        fill = jnp.arange(pad, dtype=jnp.int32)
        src = jnp.concatenate([src, fill % n])
        dst = jnp.concatenate([dst, n + fill % (n_pad - n)])
    src = src.reshape(e_pad // CHUNK, CHUNK)
    dst = dst.reshape(e_pad // CHUNK, CHUNK)
    ones = jnp.ones((n, D_SC - D_HALF), x.dtype)
    xh = jnp.stack([
        jnp.concatenate([x[:, :D_HALF], ones], axis=1),
        jnp.concatenate([x[:, D_HALF:D_IN], ones], axis=1),
    ])

    agg2 = _sc_aggregate(chunks_per_tile, n, n_pad)(xh, src, dst)
    return _tc_combine(x, agg2, W, b, n)
